# baseline (speedup 1.0000x reference)
import jax
import jax.numpy as jnp
from jax.experimental import pallas as pl


def _add_kernel(d_ref, p_ref, o_ref):
    o_ref[...] = d_ref[...] + p_ref[...]


def kernel(data, pos_emb_weight):
    B, S, E = data.shape
    W = S * E
    d2 = data.reshape(B, W)
    p2 = pos_emb_weight[:S].reshape(1, W)
    BLK = 128
    out = pl.pallas_call(
        _add_kernel,
        grid=(B // BLK,),
        in_specs=[
            pl.BlockSpec((BLK, W), lambda i: (i, 0)),
            pl.BlockSpec((1, W), lambda i: (0, 0)),
        ],
        out_specs=pl.BlockSpec((BLK, W), lambda i: (i, 0)),
        out_shape=jax.ShapeDtypeStruct((B, W), jnp.float32),
    )(d2, p2)
    return out.reshape(B, S, E)
